# baseline (speedup 1.0000x reference)
import jax
import jax.numpy as jnp
from jax.experimental import pallas as pl

N_GRAPHS = 512


def _log_softmax_body(p_ref, o_ref):
    p = p_ref[...]
    m = jnp.max(p, axis=1, keepdims=True)
    e = jnp.exp(p - m)
    o_ref[...] = p - m - jnp.log(jnp.sum(e, axis=1, keepdims=True))


def _gcn_conv(x, src, dst, ew, W, b, add_self_loops, normalize):
    n = x.shape[0]
    h = x @ W
    if normalize:
        deg = jax.ops.segment_sum(ew, dst, num_segments=n)
        dis = jnp.where(deg > 0, 1.0 / jnp.sqrt(jnp.where(deg > 0, deg, 1.0)), 0.0)
        norm = dis[src] * ew * dis[dst]
    else:
        norm = ew
    msg = norm[:, None] * jnp.take(h, src, axis=0)
    out = jax.ops.segment_sum(msg, dst, num_segments=n)
    if add_self_loops:
        out = out + h
    return out + b


def kernel(x, edge_index, batch, edge_weight, W1, b1, W2, b2, W3, b3):
    src = edge_index[0].astype(jnp.int32)
    dst = edge_index[1].astype(jnp.int32)
    batch = batch.astype(jnp.int32)
    h = jax.nn.relu(_gcn_conv(x, src, dst, edge_weight, W1, b1, True, False))
    h = jax.nn.relu(_gcn_conv(h, src, dst, edge_weight, W2, b2, False, True))
    h = jax.nn.relu(_gcn_conv(h, src, dst, edge_weight, W3, b3, True, False))
    pooled = jax.ops.segment_sum(h, batch, num_segments=N_GRAPHS)
    return pl.pallas_call(
        _log_softmax_body,
        out_shape=jax.ShapeDtypeStruct(pooled.shape, pooled.dtype),
    )(pooled)


# speedup vs baseline: 67.8199x; 67.8199x over previous
import functools

import jax
import jax.numpy as jnp
from jax import lax
from jax.experimental import pallas as pl
from jax.experimental.pallas import tpu as pltpu
from jax.experimental.pallas import tpu_sc as plsc

N = 100000
E = 6400000
G = 512
F = 16
NC = 2
NS = 16
NW = NC * NS
SLEN = 128
SUB = 8
CHUNK = SLEN * SUB
NCH = E // CHUNK
QMAX = -(-NCH // NW)
SA = 6256
SLAST = N - 15 * SA
ZB = 512

_mesh = plsc.VectorSubcoreMesh(core_axis_name="c", subcore_axis_name="s")


def _lane_bcast(v16, m):
  return jnp.take_along_axis(v16, jnp.full((16,), m, jnp.int32), axis=0)


@functools.partial(
    pl.kernel,
    out_type=jax.ShapeDtypeStruct((NC, N, F), jnp.float32),
    mesh=_mesh,
    compiler_params=pltpu.CompilerParams(use_tc_tiling_on_sc=False),
    scratch_types=[
        pltpu.VMEM_SHARED((N, F), jnp.float32),
    ],
)
def _edge_pass(src_hbm, dst_hbm, ew_hbm, table_hbm, out_hbm, acc_sh):
  c = lax.axis_index("c")
  s = lax.axis_index("s")
  wid = c * NS + s

  def _zero(zbuf):
    @plsc.parallel_loop(0, ZB, unroll=4)
    def _(r):
      zbuf[r, :] = jnp.zeros((F,), jnp.float32)
    base = s * SA

    @pl.loop(0, 12)
    def _zc(b):
      pltpu.sync_copy(zbuf, acc_sh.at[pl.ds(base + b * ZB, ZB), :])

    @pl.when(s < 15)
    def _():
      pltpu.sync_copy(zbuf.at[pl.ds(0, SA - 12 * ZB), :],
                      acc_sh.at[pl.ds(base + 12 * ZB, SA - 12 * ZB), :])
    @pl.when(s == 15)
    def _():
      pltpu.sync_copy(zbuf.at[pl.ds(0, SLAST - 12 * ZB), :],
                      acc_sh.at[pl.ds(base + 12 * ZB, SLAST - 12 * ZB), :])

  pl.run_scoped(_zero, pltpu.VMEM((ZB, F), jnp.float32))
  plsc.subcore_barrier()

  def _run(src_v, dst_v, ew_v, rows_v, gsem, ssem):
    @pl.loop(0, QMAX)
    def _chunk(q):
      cid = q * NW + wid

      @pl.when(cid < NCH)
      def _():
        rbase = cid * SUB
        pltpu.sync_copy(src_hbm.at[pl.ds(rbase, SUB), :], src_v)
        pltpu.sync_copy(dst_hbm.at[pl.ds(rbase, SUB), :], dst_v)
        pltpu.sync_copy(ew_hbm.at[pl.ds(rbase, SUB), :], ew_v)

        descs = []
        for j in range(SUB):
          descs.append(pltpu.async_copy(
              table_hbm.at[src_v.at[j]],
              rows_v.at[pl.ds(j * SLEN, SLEN), :], gsem))
        for d in descs:
          d.wait()

        @plsc.parallel_loop(0, CHUNK // 16, unroll=2)
        def _mul(g):
          j = lax.shift_right_logical(g, 3)
          k = lax.bitwise_and(g, 7)
          ew16 = ew_v[j, pl.ds(k * 16, 16)]
          for m in range(16):
            r = g * 16 + m
            rows_v[r, :] = rows_v[r, :] * _lane_bcast(ew16, m)

        descs2 = []
        for j in range(SUB):
          descs2.append(pltpu.async_copy(
              rows_v.at[pl.ds(j * SLEN, SLEN), :],
              acc_sh.at[dst_v.at[j]], ssem, add=True))
        for d in descs2:
          d.wait()

  pl.run_scoped(
      _run,
      pltpu.VMEM((SUB, SLEN), jnp.int32),
      pltpu.VMEM((SUB, SLEN), jnp.int32),
      pltpu.VMEM((SUB, SLEN), jnp.float32),
      pltpu.VMEM((CHUNK, F), jnp.float32),
      pltpu.SemaphoreType.DMA,
      pltpu.SemaphoreType.DMA,
  )

  plsc.subcore_barrier()

  @pl.loop(0, 12)
  def _co(b):
    pltpu.sync_copy(acc_sh.at[pl.ds(s * SA + b * ZB, ZB), :],
                    out_hbm.at[c, pl.ds(s * SA + b * ZB, ZB), :])

  @pl.when(s < 15)
  def _():
    pltpu.sync_copy(acc_sh.at[pl.ds(s * SA + 12 * ZB, SA - 12 * ZB), :],
                    out_hbm.at[c, pl.ds(s * SA + 12 * ZB, SA - 12 * ZB), :])
  @pl.when(s == 15)
  def _():
    pltpu.sync_copy(acc_sh.at[pl.ds(15 * SA + 12 * ZB, SLAST - 12 * ZB), :],
                    out_hbm.at[c, pl.ds(15 * SA + 12 * ZB, SLAST - 12 * ZB), :])


PB = 128
PBLK = 25
NP = NW * PBLK * PB


@functools.partial(
    pl.kernel,
    out_type=jax.ShapeDtypeStruct((NC, G, F), jnp.float32),
    mesh=_mesh,
    compiler_params=pltpu.CompilerParams(use_tc_tiling_on_sc=False),
    scratch_types=[
        pltpu.VMEM((PBLK, PB), jnp.int32),
        pltpu.VMEM((PB, F), jnp.float32),
        pltpu.VMEM_SHARED((G, F), jnp.float32),
    ],
)
def _pool(table_hbm, batch_hbm, out_hbm, bidx_v, prow_v, pool_sh):
  c = lax.axis_index("c")
  s = lax.axis_index("s")
  wid = c * NS + s

  @pl.when(s == 0)
  def _():
    def _zero(zbuf):
      @plsc.parallel_loop(0, G, unroll=4)
      def _(r):
        zbuf[r, :] = jnp.zeros((F,), jnp.float32)
      pltpu.sync_copy(zbuf, pool_sh)
    pl.run_scoped(_zero, pltpu.VMEM((G, F), jnp.float32))
  plsc.subcore_barrier()

  pltpu.sync_copy(batch_hbm.at[wid], bidx_v)

  @pl.loop(0, PBLK)
  def _blk(k):
    pltpu.sync_copy(table_hbm.at[pl.ds((wid * PBLK + k) * PB, PB), :], prow_v)
    pltpu.sync_copy(prow_v, pool_sh.at[bidx_v.at[k]], add=True)

  plsc.subcore_barrier()

  @pl.when(s == 0)
  def _():
    pltpu.sync_copy(pool_sh, out_hbm.at[c])


BR = 10000
GRID = N // BR


def _dense1_body(x_ref, w_ref, a_ref, o_ref):
  o_ref[...] = (jnp.dot(x_ref[...], w_ref[...],
                        preferred_element_type=jnp.float32) + a_ref[...])


def _dense1(x, w1p, e7):
  return pl.pallas_call(
      _dense1_body,
      grid=(GRID,),
      in_specs=[
          pl.BlockSpec((BR, 3), lambda i: (i, 0)),
          pl.BlockSpec((3, F), lambda i: (0, 0)),
          pl.BlockSpec((1, F), lambda i: (0, 0)),
      ],
      out_specs=pl.BlockSpec((BR, F), lambda i: (i, 0)),
      out_shape=jax.ShapeDtypeStruct((N, F), jnp.float32),
  )(x, w1p, e7)


def _dense2_body(p_ref, t1_ref, w2_ref, b1_ref, t2_ref, dis_ref):
  acc = p_ref[0] + p_ref[1]
  deg = acc[:, 7:8]
  dis = jnp.where(deg > 0, lax.rsqrt(jnp.where(deg > 0, deg, 1.0)), 0.0)
  out1 = jnp.maximum(acc + t1_ref[...] + b1_ref[...], 0.0)
  t2_ref[...] = dis * jnp.dot(out1, w2_ref[...],
                              preferred_element_type=jnp.float32)
  dis_ref[...] = dis


def _dense2(p1, t1, w2p, b1p):
  return pl.pallas_call(
      _dense2_body,
      grid=(GRID,),
      in_specs=[
          pl.BlockSpec((2, BR, F), lambda i: (0, i, 0)),
          pl.BlockSpec((BR, F), lambda i: (i, 0)),
          pl.BlockSpec((F, F), lambda i: (0, 0)),
          pl.BlockSpec((1, F), lambda i: (0, 0)),
      ],
      out_specs=[
          pl.BlockSpec((BR, F), lambda i: (i, 0)),
          pl.BlockSpec((BR, 1), lambda i: (i, 0)),
      ],
      out_shape=[
          jax.ShapeDtypeStruct((N, F), jnp.float32),
          jax.ShapeDtypeStruct((N, 1), jnp.float32),
      ],
  )(p1, t1, w2p, b1p)


def _dense3_body(p_ref, dis_ref, w3_ref, b2_ref, t3_ref):
  acc = p_ref[0] + p_ref[1]
  out2 = jnp.maximum(dis_ref[...] * acc + b2_ref[...], 0.0)
  t3_ref[...] = jnp.dot(out2, w3_ref[...], preferred_element_type=jnp.float32)


def _dense3(p2, dis, w3p, b2p):
  return pl.pallas_call(
      _dense3_body,
      grid=(GRID,),
      in_specs=[
          pl.BlockSpec((2, BR, F), lambda i: (0, i, 0)),
          pl.BlockSpec((BR, 1), lambda i: (i, 0)),
          pl.BlockSpec((F, F), lambda i: (0, 0)),
          pl.BlockSpec((1, F), lambda i: (0, 0)),
      ],
      out_specs=pl.BlockSpec((BR, F), lambda i: (i, 0)),
      out_shape=jax.ShapeDtypeStruct((N, F), jnp.float32),
  )(p2, dis, w3p, b2p)


def _dense4_body(p_ref, t3_ref, b3_ref, o_ref):
  o_ref[...] = jnp.maximum(p_ref[0] + p_ref[1] + t3_ref[...] + b3_ref[...],
                           0.0)


def _dense4(p3, t3, b3p):
  return pl.pallas_call(
      _dense4_body,
      grid=(GRID,),
      in_specs=[
          pl.BlockSpec((2, BR, F), lambda i: (0, i, 0)),
          pl.BlockSpec((BR, F), lambda i: (i, 0)),
          pl.BlockSpec((1, F), lambda i: (0, 0)),
      ],
      out_specs=pl.BlockSpec((BR, F), lambda i: (i, 0)),
      out_shape=jax.ShapeDtypeStruct((N, F), jnp.float32),
  )(p3, t3, b3p)


def _final_body(pp_ref, o_ref):
  pooled = pp_ref[0] + pp_ref[1]
  col = lax.broadcasted_iota(jnp.int32, (G, F), 1)
  neg = jnp.where(col < 6, pooled, -jnp.inf)
  m = jnp.max(neg, axis=1, keepdims=True)
  e = jnp.where(col < 6, jnp.exp(neg - m), 0.0)
  lse = jnp.log(jnp.sum(e, axis=1, keepdims=True))
  o_ref[...] = (pooled - m - lse)[:, :6]


def _final(pp):
  return pl.pallas_call(
      _final_body,
      out_shape=jax.ShapeDtypeStruct((G, 6), jnp.float32),
  )(pp)


def kernel(x, edge_index, batch, edge_weight, W1, b1, W2, b2, W3, b3):
  src = edge_index[0].astype(jnp.int32).reshape(E // SLEN, SLEN)
  dst = edge_index[1].astype(jnp.int32).reshape(E // SLEN, SLEN)
  ew2 = edge_weight.reshape(E // SLEN, SLEN)
  batchi = jnp.pad(batch.astype(jnp.int32), (0, NP - N),
                   constant_values=G - 1).reshape(NW, PBLK, PB)

  w1p = jnp.zeros((3, F), jnp.float32).at[:, :7].set(W1)
  e7 = jnp.zeros((1, F), jnp.float32).at[0, 7].set(1.0)
  b1p = jnp.zeros((1, F), jnp.float32).at[0, :7].set(b1)
  w2p = jnp.zeros((F, F), jnp.float32).at[:7, :9].set(W2)
  b2p = jnp.zeros((1, F), jnp.float32).at[0, :9].set(b2)
  w3p = jnp.zeros((F, F), jnp.float32).at[:9, :6].set(W3)
  b3p = jnp.zeros((1, F), jnp.float32).at[0, :6].set(b3)

  t1 = _dense1(x, w1p, e7)
  p1 = _edge_pass(src, dst, ew2, t1)
  t2, dis = _dense2(p1, t1, w2p, b1p)
  p2 = _edge_pass(src, dst, ew2, t2)
  t3 = _dense3(p2, dis, w3p, b2p)
  p3 = _edge_pass(src, dst, ew2, t3)
  out3 = _dense4(p3, t3, b3p)
  out3p = jnp.pad(out3, ((0, NP - N), (0, 0)))
  pp = _pool(out3p, batchi)
  return _final(pp)


# speedup vs baseline: 103.3812x; 1.5243x over previous
import functools

import jax
import jax.numpy as jnp
from jax import lax
from jax.experimental import pallas as pl
from jax.experimental.pallas import tpu as pltpu
from jax.experimental.pallas import tpu_sc as plsc

N = 100000
E = 6400000
G = 512
F = 16
NC = 2
NS = 16
NW = NC * NS
SLEN = 64
SUB = 8
CHUNK = SLEN * SUB
NCH = E // CHUNK
QMAX = (-(-NCH // NW) + 1) // 2 * 2
GSH = 2
SA = 6256
SLAST = N - 15 * SA
ZB = 128
CB = 512

_mesh = plsc.VectorSubcoreMesh(core_axis_name="c", subcore_axis_name="s")


def _lane_bcast(v16, m):
  return jnp.take_along_axis(v16, jnp.full((16,), m, jnp.int32), axis=0)


@functools.partial(
    pl.kernel,
    out_type=jax.ShapeDtypeStruct((NC, N, F), jnp.float32),
    mesh=_mesh,
    compiler_params=pltpu.CompilerParams(use_tc_tiling_on_sc=False),
    scratch_types=[
        pltpu.VMEM_SHARED((N, F), jnp.float32),
    ],
)
def _edge_pass(src_hbm, dst_hbm, ew_hbm, table_hbm, out_hbm, acc_sh):
  c = lax.axis_index("c")
  s = lax.axis_index("s")
  wid = c * NS + s

  def _zero(zbuf):
    @plsc.parallel_loop(0, ZB, unroll=4)
    def _(r):
      zbuf[r, :] = jnp.zeros((F,), jnp.float32)
    base = s * SA

    @pl.loop(0, 48)
    def _zc(b):
      pltpu.sync_copy(zbuf, acc_sh.at[pl.ds(base + b * ZB, ZB), :])

    @pl.when(s < 15)
    def _():
      pltpu.sync_copy(zbuf.at[pl.ds(0, SA - 48 * ZB), :],
                      acc_sh.at[pl.ds(base + 48 * ZB, SA - 48 * ZB), :])
    @pl.when(s == 15)
    def _():
      pltpu.sync_copy(zbuf.at[pl.ds(0, SLAST - 48 * ZB), :],
                      acc_sh.at[pl.ds(base + 48 * ZB, SLAST - 48 * ZB), :])

  pl.run_scoped(_zero, pltpu.VMEM((ZB, F), jnp.float32))
  plsc.subcore_barrier()

  def _run(srcA, dstA, ewA, rowsA, isA, gsA, ssA,
           srcB, dstB, ewB, rowsB, isB, gsB, ssB):
    def _valid(q):
      cid = q * NW + wid
      return (cid >= 0) & (cid < NCH)

    def _load_fire(q, srcv, dstv, ewv, rowsv, isem, gsem):
      @pl.when(_valid(q))
      def _():
        rbase = (q * NW + wid) * SUB
        pltpu.async_copy(src_hbm.at[pl.ds(rbase, SUB), :], srcv, isem)
        pltpu.async_copy(dst_hbm.at[pl.ds(rbase, SUB), :], dstv, isem)
        pltpu.async_copy(ew_hbm.at[pl.ds(rbase, SUB), :], ewv, isem)
        pltpu.make_async_copy(src_hbm.at[pl.ds(rbase, SUB), :], srcv,
                              isem).wait()
        pltpu.make_async_copy(dst_hbm.at[pl.ds(rbase, SUB), :], dstv,
                              isem).wait()
        pltpu.make_async_copy(ew_hbm.at[pl.ds(rbase, SUB), :], ewv,
                              isem).wait()
        for j in range(SUB):
          pltpu.async_copy(table_hbm.at[srcv.at[j]],
                           rowsv.at[pl.ds(j * SLEN, SLEN), :], gsem)

    def _mul_scatter(q, srcv, dstv, ewv, rowsv, gsem, ssem):
      @pl.when(_valid(q))
      def _():
        for j in range(SUB):
          pltpu.make_async_copy(table_hbm.at[srcv.at[j]],
                                rowsv.at[pl.ds(j * SLEN, SLEN), :],
                                gsem).wait()

        @plsc.parallel_loop(0, CHUNK // 16, unroll=2)
        def _mul(g):
          j = lax.shift_right_logical(g, GSH)
          k = lax.bitwise_and(g, (1 << GSH) - 1)
          ew16 = ewv[j, pl.ds(k * 16, 16)]
          for m in range(16):
            r = g * 16 + m
            rowsv[r, :] = rowsv[r, :] * _lane_bcast(ew16, m)

        for j in range(SUB):
          pltpu.async_copy(rowsv.at[pl.ds(j * SLEN, SLEN), :],
                           acc_sh.at[dstv.at[j]], ssem, add=True)

    def _drain_scatter(q, dstv, rowsv, ssem):
      @pl.when(_valid(q))
      def _():
        for j in range(SUB):
          pltpu.make_async_copy(rowsv.at[pl.ds(j * SLEN, SLEN), :],
                                acc_sh.at[dstv.at[j]], ssem).wait()

    _load_fire(0, srcA, dstA, ewA, rowsA, isA, gsA)

    @pl.loop(0, QMAX // 2)
    def _chunk(t):
      qa = 2 * t
      _drain_scatter(qa - 1, dstB, rowsB, ssB)
      _load_fire(qa + 1, srcB, dstB, ewB, rowsB, isB, gsB)
      _mul_scatter(qa, srcA, dstA, ewA, rowsA, gsA, ssA)
      _drain_scatter(qa, dstA, rowsA, ssA)
      _load_fire(qa + 2, srcA, dstA, ewA, rowsA, isA, gsA)
      _mul_scatter(qa + 1, srcB, dstB, ewB, rowsB, gsB, ssB)

    _drain_scatter(QMAX - 1, dstB, rowsB, ssB)

  pl.run_scoped(
      _run,
      pltpu.VMEM((SUB, SLEN), jnp.int32),
      pltpu.VMEM((SUB, SLEN), jnp.int32),
      pltpu.VMEM((SUB, SLEN), jnp.float32),
      pltpu.VMEM((CHUNK, F), jnp.float32),
      pltpu.SemaphoreType.DMA,
      pltpu.SemaphoreType.DMA,
      pltpu.SemaphoreType.DMA,
      pltpu.VMEM((SUB, SLEN), jnp.int32),
      pltpu.VMEM((SUB, SLEN), jnp.int32),
      pltpu.VMEM((SUB, SLEN), jnp.float32),
      pltpu.VMEM((CHUNK, F), jnp.float32),
      pltpu.SemaphoreType.DMA,
      pltpu.SemaphoreType.DMA,
      pltpu.SemaphoreType.DMA,
  )

  plsc.subcore_barrier()

  @pl.loop(0, 12)
  def _co(b):
    pltpu.sync_copy(acc_sh.at[pl.ds(s * SA + b * CB, CB), :],
                    out_hbm.at[c, pl.ds(s * SA + b * CB, CB), :])

  @pl.when(s < 15)
  def _():
    pltpu.sync_copy(acc_sh.at[pl.ds(s * SA + 12 * CB, SA - 12 * CB), :],
                    out_hbm.at[c, pl.ds(s * SA + 12 * CB, SA - 12 * CB), :])
  @pl.when(s == 15)
  def _():
    pltpu.sync_copy(acc_sh.at[pl.ds(15 * SA + 12 * CB, SLAST - 12 * CB), :],
                    out_hbm.at[c, pl.ds(15 * SA + 12 * CB, SLAST - 12 * CB), :])


PB = 128
PBLK = 25
NP = NW * PBLK * PB


@functools.partial(
    pl.kernel,
    out_type=jax.ShapeDtypeStruct((NC, G, F), jnp.float32),
    mesh=_mesh,
    compiler_params=pltpu.CompilerParams(use_tc_tiling_on_sc=False),
    scratch_types=[
        pltpu.VMEM((PBLK, PB), jnp.int32),
        pltpu.VMEM((PB, F), jnp.float32),
        pltpu.VMEM_SHARED((G, F), jnp.float32),
    ],
)
def _pool(table_hbm, batch_hbm, out_hbm, bidx_v, prow_v, pool_sh):
  c = lax.axis_index("c")
  s = lax.axis_index("s")
  wid = c * NS + s

  @pl.when(s == 0)
  def _():
    def _zero(zbuf):
      @plsc.parallel_loop(0, G, unroll=4)
      def _(r):
        zbuf[r, :] = jnp.zeros((F,), jnp.float32)
      pltpu.sync_copy(zbuf, pool_sh)
    pl.run_scoped(_zero, pltpu.VMEM((G, F), jnp.float32))
  plsc.subcore_barrier()

  pltpu.sync_copy(batch_hbm.at[wid], bidx_v)

  @pl.loop(0, PBLK)
  def _blk(k):
    pltpu.sync_copy(table_hbm.at[pl.ds((wid * PBLK + k) * PB, PB), :], prow_v)
    pltpu.sync_copy(prow_v, pool_sh.at[bidx_v.at[k]], add=True)

  plsc.subcore_barrier()

  @pl.when(s == 0)
  def _():
    pltpu.sync_copy(pool_sh, out_hbm.at[c])


BR = 10000
GRID = N // BR


def _dense1_body(x_ref, w_ref, a_ref, o_ref):
  o_ref[...] = (jnp.dot(x_ref[...], w_ref[...],
                        preferred_element_type=jnp.float32) + a_ref[...])


def _dense1(x, w1p, e7):
  return pl.pallas_call(
      _dense1_body,
      grid=(GRID,),
      in_specs=[
          pl.BlockSpec((BR, 3), lambda i: (i, 0)),
          pl.BlockSpec((3, F), lambda i: (0, 0)),
          pl.BlockSpec((1, F), lambda i: (0, 0)),
      ],
      out_specs=pl.BlockSpec((BR, F), lambda i: (i, 0)),
      out_shape=jax.ShapeDtypeStruct((N, F), jnp.float32),
  )(x, w1p, e7)


def _dense2_body(p_ref, t1_ref, w2_ref, b1_ref, t2_ref, dis_ref):
  acc = p_ref[0] + p_ref[1]
  deg = acc[:, 7:8]
  dis = jnp.where(deg > 0, lax.rsqrt(jnp.where(deg > 0, deg, 1.0)), 0.0)
  out1 = jnp.maximum(acc + t1_ref[...] + b1_ref[...], 0.0)
  t2_ref[...] = dis * jnp.dot(out1, w2_ref[...],
                              preferred_element_type=jnp.float32)
  dis_ref[...] = dis


def _dense2(p1, t1, w2p, b1p):
  return pl.pallas_call(
      _dense2_body,
      grid=(GRID,),
      in_specs=[
          pl.BlockSpec((2, BR, F), lambda i: (0, i, 0)),
          pl.BlockSpec((BR, F), lambda i: (i, 0)),
          pl.BlockSpec((F, F), lambda i: (0, 0)),
          pl.BlockSpec((1, F), lambda i: (0, 0)),
      ],
      out_specs=[
          pl.BlockSpec((BR, F), lambda i: (i, 0)),
          pl.BlockSpec((BR, 1), lambda i: (i, 0)),
      ],
      out_shape=[
          jax.ShapeDtypeStruct((N, F), jnp.float32),
          jax.ShapeDtypeStruct((N, 1), jnp.float32),
      ],
  )(p1, t1, w2p, b1p)


def _dense3_body(p_ref, dis_ref, w3_ref, b2_ref, t3_ref):
  acc = p_ref[0] + p_ref[1]
  out2 = jnp.maximum(dis_ref[...] * acc + b2_ref[...], 0.0)
  t3_ref[...] = jnp.dot(out2, w3_ref[...], preferred_element_type=jnp.float32)


def _dense3(p2, dis, w3p, b2p):
  return pl.pallas_call(
      _dense3_body,
      grid=(GRID,),
      in_specs=[
          pl.BlockSpec((2, BR, F), lambda i: (0, i, 0)),
          pl.BlockSpec((BR, 1), lambda i: (i, 0)),
          pl.BlockSpec((F, F), lambda i: (0, 0)),
          pl.BlockSpec((1, F), lambda i: (0, 0)),
      ],
      out_specs=pl.BlockSpec((BR, F), lambda i: (i, 0)),
      out_shape=jax.ShapeDtypeStruct((N, F), jnp.float32),
  )(p2, dis, w3p, b2p)


def _dense4_body(p_ref, t3_ref, b3_ref, o_ref):
  o_ref[...] = jnp.maximum(p_ref[0] + p_ref[1] + t3_ref[...] + b3_ref[...],
                           0.0)


def _dense4(p3, t3, b3p):
  return pl.pallas_call(
      _dense4_body,
      grid=(GRID,),
      in_specs=[
          pl.BlockSpec((2, BR, F), lambda i: (0, i, 0)),
          pl.BlockSpec((BR, F), lambda i: (i, 0)),
          pl.BlockSpec((1, F), lambda i: (0, 0)),
      ],
      out_specs=pl.BlockSpec((BR, F), lambda i: (i, 0)),
      out_shape=jax.ShapeDtypeStruct((N, F), jnp.float32),
  )(p3, t3, b3p)


def _final_body(pp_ref, o_ref):
  pooled = pp_ref[0] + pp_ref[1]
  col = lax.broadcasted_iota(jnp.int32, (G, F), 1)
  neg = jnp.where(col < 6, pooled, -jnp.inf)
  m = jnp.max(neg, axis=1, keepdims=True)
  e = jnp.where(col < 6, jnp.exp(neg - m), 0.0)
  lse = jnp.log(jnp.sum(e, axis=1, keepdims=True))
  o_ref[...] = (pooled - m - lse)[:, :6]


def _final(pp):
  return pl.pallas_call(
      _final_body,
      out_shape=jax.ShapeDtypeStruct((G, 6), jnp.float32),
  )(pp)


def kernel(x, edge_index, batch, edge_weight, W1, b1, W2, b2, W3, b3):
  src = edge_index[0].astype(jnp.int32).reshape(E // SLEN, SLEN)
  dst = edge_index[1].astype(jnp.int32).reshape(E // SLEN, SLEN)
  ew2 = edge_weight.reshape(E // SLEN, SLEN)
  batchi = jnp.pad(batch.astype(jnp.int32), (0, NP - N),
                   constant_values=G - 1).reshape(NW, PBLK, PB)

  w1p = jnp.zeros((3, F), jnp.float32).at[:, :7].set(W1)
  e7 = jnp.zeros((1, F), jnp.float32).at[0, 7].set(1.0)
  b1p = jnp.zeros((1, F), jnp.float32).at[0, :7].set(b1)
  w2p = jnp.zeros((F, F), jnp.float32).at[:7, :9].set(W2)
  b2p = jnp.zeros((1, F), jnp.float32).at[0, :9].set(b2)
  w3p = jnp.zeros((F, F), jnp.float32).at[:9, :6].set(W3)
  b3p = jnp.zeros((1, F), jnp.float32).at[0, :6].set(b3)

  t1 = _dense1(x, w1p, e7)
  p1 = _edge_pass(src, dst, ew2, t1)
  t2, dis = _dense2(p1, t1, w2p, b1p)
  p2 = _edge_pass(src, dst, ew2, t2)
  t3 = _dense3(p2, dis, w3p, b2p)
  p3 = _edge_pass(src, dst, ew2, t3)
  out3 = _dense4(p3, t3, b3p)
  out3p = jnp.pad(out3, ((0, NP - N), (0, 0)))
  pp = _pool(out3p, batchi)
  return _final(pp)


# speedup vs baseline: 109.2832x; 1.0571x over previous
import functools

import jax
import jax.numpy as jnp
from jax import lax
from jax.experimental import pallas as pl
from jax.experimental.pallas import tpu as pltpu
from jax.experimental.pallas import tpu_sc as plsc

N = 100000
E = 6400000
G = 512
F = 16
NC = 2
NS = 16
NW = NC * NS
SLEN = 64
SUB = 8
CHUNK = SLEN * SUB
NCH = E // CHUNK
QMAX = (-(-NCH // NW) + 1) // 2 * 2
GSH = 2
SA = 6256
SLAST = N - 15 * SA
ZB = 128
CB = 256

_mesh = plsc.VectorSubcoreMesh(core_axis_name="c", subcore_axis_name="s")


def _lane_bcast(v16, m):
  return jnp.take_along_axis(v16, jnp.full((16,), m, jnp.int32), axis=0)


@functools.partial(
    pl.kernel,
    out_type=jax.ShapeDtypeStruct((NC, N, F), jnp.float32),
    mesh=_mesh,
    compiler_params=pltpu.CompilerParams(use_tc_tiling_on_sc=False),
    scratch_types=[
        pltpu.VMEM_SHARED((N, F), jnp.float32),
    ],
)
def _edge_pass(src_hbm, dst_hbm, ew_hbm, table_hbm, out_hbm, acc_sh):
  c = lax.axis_index("c")
  s = lax.axis_index("s")
  wid = c * NS + s

  def _zero(zbuf):
    @plsc.parallel_loop(0, ZB, unroll=4)
    def _(r):
      zbuf[r, :] = jnp.zeros((F,), jnp.float32)
    base = s * SA

    @pl.loop(0, 48)
    def _zc(b):
      pltpu.sync_copy(zbuf, acc_sh.at[pl.ds(base + b * ZB, ZB), :])

    @pl.when(s < 15)
    def _():
      pltpu.sync_copy(zbuf.at[pl.ds(0, SA - 48 * ZB), :],
                      acc_sh.at[pl.ds(base + 48 * ZB, SA - 48 * ZB), :])
    @pl.when(s == 15)
    def _():
      pltpu.sync_copy(zbuf.at[pl.ds(0, SLAST - 48 * ZB), :],
                      acc_sh.at[pl.ds(base + 48 * ZB, SLAST - 48 * ZB), :])

  pl.run_scoped(_zero, pltpu.VMEM((ZB, F), jnp.float32))
  plsc.subcore_barrier()

  def _run(srcA, dstA, ewA, rowsA, isA, gsA, ssA,
           srcB, dstB, ewB, rowsB, isB, gsB, ssB):
    def _valid(q):
      cid = q * NW + wid
      return (cid >= 0) & (cid < NCH)

    def _load_fire(q, srcv, dstv, ewv, rowsv, isem, gsem):
      @pl.when(_valid(q))
      def _():
        rbase = (q * NW + wid) * SUB
        pltpu.async_copy(src_hbm.at[pl.ds(rbase, SUB), :], srcv, isem)
        pltpu.async_copy(dst_hbm.at[pl.ds(rbase, SUB), :], dstv, isem)
        pltpu.async_copy(ew_hbm.at[pl.ds(rbase, SUB), :], ewv, isem)
        pltpu.make_async_copy(src_hbm.at[pl.ds(rbase, SUB), :], srcv,
                              isem).wait()
        pltpu.make_async_copy(dst_hbm.at[pl.ds(rbase, SUB), :], dstv,
                              isem).wait()
        pltpu.make_async_copy(ew_hbm.at[pl.ds(rbase, SUB), :], ewv,
                              isem).wait()
        for j in range(SUB):
          pltpu.async_copy(table_hbm.at[srcv.at[j]],
                           rowsv.at[pl.ds(j * SLEN, SLEN), :], gsem)

    def _mul_scatter(q, srcv, dstv, ewv, rowsv, gsem, ssem):
      @pl.when(_valid(q))
      def _():
        for j in range(SUB):
          pltpu.make_async_copy(table_hbm.at[srcv.at[j]],
                                rowsv.at[pl.ds(j * SLEN, SLEN), :],
                                gsem).wait()

        @plsc.parallel_loop(0, CHUNK // 16, unroll=2)
        def _mul(g):
          j = lax.shift_right_logical(g, GSH)
          k = lax.bitwise_and(g, (1 << GSH) - 1)
          ew16 = ewv[j, pl.ds(k * 16, 16)]
          for m in range(16):
            r = g * 16 + m
            rowsv[r, :] = rowsv[r, :] * _lane_bcast(ew16, m)

        for j in range(SUB):
          pltpu.async_copy(rowsv.at[pl.ds(j * SLEN, SLEN), :],
                           acc_sh.at[dstv.at[j]], ssem, add=True)

    def _drain_scatter(q, dstv, rowsv, ssem):
      @pl.when(_valid(q))
      def _():
        for j in range(SUB):
          pltpu.make_async_copy(rowsv.at[pl.ds(j * SLEN, SLEN), :],
                                acc_sh.at[dstv.at[j]], ssem).wait()

    _load_fire(0, srcA, dstA, ewA, rowsA, isA, gsA)

    @pl.loop(0, QMAX // 2)
    def _chunk(t):
      qa = 2 * t
      _drain_scatter(qa - 1, dstB, rowsB, ssB)
      _load_fire(qa + 1, srcB, dstB, ewB, rowsB, isB, gsB)
      _mul_scatter(qa, srcA, dstA, ewA, rowsA, gsA, ssA)
      _drain_scatter(qa, dstA, rowsA, ssA)
      _load_fire(qa + 2, srcA, dstA, ewA, rowsA, isA, gsA)
      _mul_scatter(qa + 1, srcB, dstB, ewB, rowsB, gsB, ssB)

    _drain_scatter(QMAX - 1, dstB, rowsB, ssB)

  pl.run_scoped(
      _run,
      pltpu.VMEM((SUB, SLEN), jnp.int32),
      pltpu.VMEM((SUB, SLEN), jnp.int32),
      pltpu.VMEM((SUB, SLEN), jnp.float32),
      pltpu.VMEM((CHUNK, F), jnp.float32),
      pltpu.SemaphoreType.DMA,
      pltpu.SemaphoreType.DMA,
      pltpu.SemaphoreType.DMA,
      pltpu.VMEM((SUB, SLEN), jnp.int32),
      pltpu.VMEM((SUB, SLEN), jnp.int32),
      pltpu.VMEM((SUB, SLEN), jnp.float32),
      pltpu.VMEM((CHUNK, F), jnp.float32),
      pltpu.SemaphoreType.DMA,
      pltpu.SemaphoreType.DMA,
      pltpu.SemaphoreType.DMA,
  )

  plsc.subcore_barrier()

  @pl.loop(0, 24)
  def _co(b):
    pltpu.sync_copy(acc_sh.at[pl.ds(s * SA + b * CB, CB), :],
                    out_hbm.at[c, pl.ds(s * SA + b * CB, CB), :])

  @pl.when(s < 15)
  def _():
    pltpu.sync_copy(acc_sh.at[pl.ds(s * SA + 24 * CB, SA - 24 * CB), :],
                    out_hbm.at[c, pl.ds(s * SA + 24 * CB, SA - 24 * CB), :])
  @pl.when(s == 15)
  def _():
    pltpu.sync_copy(acc_sh.at[pl.ds(15 * SA + 24 * CB, SLAST - 24 * CB), :],
                    out_hbm.at[c, pl.ds(15 * SA + 24 * CB, SLAST - 24 * CB), :])


PB = 128
PBLK = 25
NP = NW * PBLK * PB
TAIL_BASE = 31 * PBLK * PB + 6 * PB
TAIL = N - TAIL_BASE


@functools.partial(
    pl.kernel,
    out_type=jax.ShapeDtypeStruct((NC, G, F), jnp.float32),
    mesh=_mesh,
    compiler_params=pltpu.CompilerParams(use_tc_tiling_on_sc=False),
    scratch_types=[
        pltpu.VMEM((PBLK, PB), jnp.int32),
        pltpu.VMEM((1, TAIL), jnp.int32),
        pltpu.VMEM((PB, F), jnp.float32),
        pltpu.VMEM((PB, F), jnp.float32),
        pltpu.VMEM((PB, F), jnp.float32),
        pltpu.VMEM((1, F), jnp.float32),
        pltpu.VMEM_SHARED((G, F), jnp.float32),
    ],
)
def _pool(p3_hbm, t3_hbm, b3_hbm, batch_hbm, out_hbm,
          bidx_v, tidx_v, prow_v, paux_v, taux_v, bv, pool_sh):
  c = lax.axis_index("c")
  s = lax.axis_index("s")
  wid = c * NS + s

  @pl.when(s == 0)
  def _():
    def _zero(zbuf):
      @plsc.parallel_loop(0, G, unroll=4)
      def _(r):
        zbuf[r, :] = jnp.zeros((F,), jnp.float32)
      pltpu.sync_copy(zbuf, pool_sh)
    pl.run_scoped(_zero, pltpu.VMEM((G, F), jnp.float32))
  plsc.subcore_barrier()

  pltpu.sync_copy(batch_hbm.at[wid], bidx_v)
  pltpu.sync_copy(b3_hbm, bv)
  nblk = jnp.where(wid < 31, PBLK, 6)

  def _epilogue(nrows, nb, idx):
    pltpu.sync_copy(p3_hbm.at[0, pl.ds(nb, nrows), :],
                    prow_v.at[pl.ds(0, nrows), :])
    pltpu.sync_copy(p3_hbm.at[1, pl.ds(nb, nrows), :],
                    paux_v.at[pl.ds(0, nrows), :])
    pltpu.sync_copy(t3_hbm.at[pl.ds(nb, nrows), :],
                    taux_v.at[pl.ds(0, nrows), :])

    @plsc.parallel_loop(0, nrows, unroll=4)
    def _(r):
      prow_v[r, :] = jnp.maximum(
          prow_v[r, :] + paux_v[r, :] + taux_v[r, :] + bv[0, :], 0.0)

    pltpu.sync_copy(prow_v.at[pl.ds(0, nrows), :], pool_sh.at[idx], add=True)

  @pl.loop(0, PBLK)
  def _blk(k):
    @pl.when(k < nblk)
    def _():
      _epilogue(PB, wid * PBLK * PB + k * PB, bidx_v.at[k])

  @pl.when(wid == 31)
  def _():
    pltpu.sync_copy(batch_hbm.at[31, pl.ds(6, 1), pl.ds(0, TAIL)], tidx_v)
    _epilogue(TAIL, TAIL_BASE, tidx_v.at[0])

  plsc.subcore_barrier()

  @pl.when(s == 0)
  def _():
    pltpu.sync_copy(pool_sh, out_hbm.at[c])


BR = 10000
GRID = N // BR


def _dense1_body(x_ref, w_ref, a_ref, o_ref):
  o_ref[...] = (jnp.dot(x_ref[...], w_ref[...],
                        preferred_element_type=jnp.float32) + a_ref[...])


def _dense1(x, w1p, e7):
  return pl.pallas_call(
      _dense1_body,
      grid=(GRID,),
      in_specs=[
          pl.BlockSpec((BR, 3), lambda i: (i, 0)),
          pl.BlockSpec((3, F), lambda i: (0, 0)),
          pl.BlockSpec((1, F), lambda i: (0, 0)),
      ],
      out_specs=pl.BlockSpec((BR, F), lambda i: (i, 0)),
      out_shape=jax.ShapeDtypeStruct((N, F), jnp.float32),
  )(x, w1p, e7)


def _dense2_body(p_ref, t1_ref, w2_ref, b1_ref, t2_ref, dis_ref):
  acc = p_ref[0] + p_ref[1]
  deg = acc[:, 7:8]
  dis = jnp.where(deg > 0, lax.rsqrt(jnp.where(deg > 0, deg, 1.0)), 0.0)
  out1 = jnp.maximum(acc + t1_ref[...] + b1_ref[...], 0.0)
  t2_ref[...] = dis * jnp.dot(out1, w2_ref[...],
                              preferred_element_type=jnp.float32)
  dis_ref[...] = dis


def _dense2(p1, t1, w2p, b1p):
  return pl.pallas_call(
      _dense2_body,
      grid=(GRID,),
      in_specs=[
          pl.BlockSpec((2, BR, F), lambda i: (0, i, 0)),
          pl.BlockSpec((BR, F), lambda i: (i, 0)),
          pl.BlockSpec((F, F), lambda i: (0, 0)),
          pl.BlockSpec((1, F), lambda i: (0, 0)),
      ],
      out_specs=[
          pl.BlockSpec((BR, F), lambda i: (i, 0)),
          pl.BlockSpec((BR, 1), lambda i: (i, 0)),
      ],
      out_shape=[
          jax.ShapeDtypeStruct((N, F), jnp.float32),
          jax.ShapeDtypeStruct((N, 1), jnp.float32),
      ],
  )(p1, t1, w2p, b1p)


def _dense3_body(p_ref, dis_ref, w3_ref, b2_ref, t3_ref):
  acc = p_ref[0] + p_ref[1]
  out2 = jnp.maximum(dis_ref[...] * acc + b2_ref[...], 0.0)
  t3_ref[...] = jnp.dot(out2, w3_ref[...], preferred_element_type=jnp.float32)


def _dense3(p2, dis, w3p, b2p):
  return pl.pallas_call(
      _dense3_body,
      grid=(GRID,),
      in_specs=[
          pl.BlockSpec((2, BR, F), lambda i: (0, i, 0)),
          pl.BlockSpec((BR, 1), lambda i: (i, 0)),
          pl.BlockSpec((F, F), lambda i: (0, 0)),
          pl.BlockSpec((1, F), lambda i: (0, 0)),
      ],
      out_specs=pl.BlockSpec((BR, F), lambda i: (i, 0)),
      out_shape=jax.ShapeDtypeStruct((N, F), jnp.float32),
  )(p2, dis, w3p, b2p)


def _dense4_body(p_ref, t3_ref, b3_ref, o_ref):
  o_ref[...] = jnp.maximum(p_ref[0] + p_ref[1] + t3_ref[...] + b3_ref[...],
                           0.0)


def _dense4(p3, t3, b3p):
  return pl.pallas_call(
      _dense4_body,
      grid=(GRID,),
      in_specs=[
          pl.BlockSpec((2, BR, F), lambda i: (0, i, 0)),
          pl.BlockSpec((BR, F), lambda i: (i, 0)),
          pl.BlockSpec((1, F), lambda i: (0, 0)),
      ],
      out_specs=pl.BlockSpec((BR, F), lambda i: (i, 0)),
      out_shape=jax.ShapeDtypeStruct((N, F), jnp.float32),
  )(p3, t3, b3p)


def _final_body(pp_ref, o_ref):
  pooled = pp_ref[0] + pp_ref[1]
  col = lax.broadcasted_iota(jnp.int32, (G, F), 1)
  neg = jnp.where(col < 6, pooled, -jnp.inf)
  m = jnp.max(neg, axis=1, keepdims=True)
  e = jnp.where(col < 6, jnp.exp(neg - m), 0.0)
  lse = jnp.log(jnp.sum(e, axis=1, keepdims=True))
  o_ref[...] = (pooled - m - lse)[:, :6]


def _final(pp):
  return pl.pallas_call(
      _final_body,
      out_shape=jax.ShapeDtypeStruct((G, 6), jnp.float32),
  )(pp)


def kernel(x, edge_index, batch, edge_weight, W1, b1, W2, b2, W3, b3):
  src = edge_index[0].astype(jnp.int32).reshape(E // SLEN, SLEN)
  dst = edge_index[1].astype(jnp.int32).reshape(E // SLEN, SLEN)
  ew2 = edge_weight.reshape(E // SLEN, SLEN)
  batchi = jnp.pad(batch.astype(jnp.int32), (0, NP - N),
                   constant_values=G - 1).reshape(NW, PBLK, PB)

  w1p = jnp.zeros((3, F), jnp.float32).at[:, :7].set(W1)
  e7 = jnp.zeros((1, F), jnp.float32).at[0, 7].set(1.0)
  b1p = jnp.zeros((1, F), jnp.float32).at[0, :7].set(b1)
  w2p = jnp.zeros((F, F), jnp.float32).at[:7, :9].set(W2)
  b2p = jnp.zeros((1, F), jnp.float32).at[0, :9].set(b2)
  w3p = jnp.zeros((F, F), jnp.float32).at[:9, :6].set(W3)
  b3p = jnp.zeros((1, F), jnp.float32).at[0, :6].set(b3)

  t1 = _dense1(x, w1p, e7)
  p1 = _edge_pass(src, dst, ew2, t1)
  t2, dis = _dense2(p1, t1, w2p, b1p)
  p2 = _edge_pass(src, dst, ew2, t2)
  t3 = _dense3(p2, dis, w3p, b2p)
  p3 = _edge_pass(src, dst, ew2, t3)
  pp = _pool(p3, t3, b3p, batchi)
  return _final(pp)


# speedup vs baseline: 121.6601x; 1.1133x over previous
import functools

import jax
import jax.numpy as jnp
from jax import lax
from jax.experimental import pallas as pl
from jax.experimental.pallas import tpu as pltpu
from jax.experimental.pallas import tpu_sc as plsc

N = 100000
E = 6400000
G = 512
F = 16
NC = 2
NS = 16
NW = NC * NS
SLEN = 64
SUB = 8
CHUNK = SLEN * SUB
NCH = E // CHUNK
QMAX = (-(-NCH // NW) + 2) // 3 * 3
GSH = 2
SA = 6256
SLAST = N - 15 * SA
ZB = 128
CB = 256

_mesh = plsc.VectorSubcoreMesh(core_axis_name="c", subcore_axis_name="s")


def _lane_bcast(v16, m):
  return jnp.take_along_axis(v16, jnp.full((16,), m, jnp.int32), axis=0)


@functools.partial(
    pl.kernel,
    out_type=jax.ShapeDtypeStruct((NC, N, F), jnp.float32),
    mesh=_mesh,
    compiler_params=pltpu.CompilerParams(use_tc_tiling_on_sc=False),
    scratch_types=[
        pltpu.VMEM_SHARED((N, F), jnp.float32),
    ],
)
def _edge_pass(src_hbm, dst_hbm, ew_hbm, table_hbm, out_hbm, acc_sh):
  c = lax.axis_index("c")
  s = lax.axis_index("s")
  wid = c * NS + s

  def _zero(zbuf):
    @plsc.parallel_loop(0, ZB, unroll=4)
    def _(r):
      zbuf[r, :] = jnp.zeros((F,), jnp.float32)
    base = s * SA

    @pl.loop(0, 48)
    def _zc(b):
      pltpu.sync_copy(zbuf, acc_sh.at[pl.ds(base + b * ZB, ZB), :])

    @pl.when(s < 15)
    def _():
      pltpu.sync_copy(zbuf.at[pl.ds(0, SA - 48 * ZB), :],
                      acc_sh.at[pl.ds(base + 48 * ZB, SA - 48 * ZB), :])
    @pl.when(s == 15)
    def _():
      pltpu.sync_copy(zbuf.at[pl.ds(0, SLAST - 48 * ZB), :],
                      acc_sh.at[pl.ds(base + 48 * ZB, SLAST - 48 * ZB), :])

  pl.run_scoped(_zero, pltpu.VMEM((ZB, F), jnp.float32))
  plsc.subcore_barrier()

  def _run(srcA, dstA, ewA, rowsA, isA, gsA, ssA,
           srcB, dstB, ewB, rowsB, isB, gsB, ssB,
           srcC, dstC, ewC, rowsC, isC, gsC, ssC):
    def _valid(q):
      cid = q * NW + wid
      return (cid >= 0) & (cid < NCH)

    def _load_fire(q, srcv, dstv, ewv, rowsv, isem, gsem):
      @pl.when(_valid(q))
      def _():
        rbase = (q * NW + wid) * SUB
        pltpu.async_copy(src_hbm.at[pl.ds(rbase, SUB), :], srcv, isem)
        pltpu.async_copy(dst_hbm.at[pl.ds(rbase, SUB), :], dstv, isem)
        pltpu.async_copy(ew_hbm.at[pl.ds(rbase, SUB), :], ewv, isem)
        pltpu.make_async_copy(src_hbm.at[pl.ds(rbase, SUB), :], srcv,
                              isem).wait()
        pltpu.make_async_copy(dst_hbm.at[pl.ds(rbase, SUB), :], dstv,
                              isem).wait()
        pltpu.make_async_copy(ew_hbm.at[pl.ds(rbase, SUB), :], ewv,
                              isem).wait()
        for j in range(SUB):
          pltpu.async_copy(table_hbm.at[srcv.at[j]],
                           rowsv.at[pl.ds(j * SLEN, SLEN), :], gsem)

    def _mul_scatter(q, srcv, dstv, ewv, rowsv, gsem, ssem):
      @pl.when(_valid(q))
      def _():
        for j in range(SUB):
          pltpu.make_async_copy(table_hbm.at[srcv.at[j]],
                                rowsv.at[pl.ds(j * SLEN, SLEN), :],
                                gsem).wait()

        @plsc.parallel_loop(0, CHUNK // 16, unroll=2)
        def _mul(g):
          j = lax.shift_right_logical(g, GSH)
          k = lax.bitwise_and(g, (1 << GSH) - 1)
          ew16 = ewv[j, pl.ds(k * 16, 16)]
          for m in range(16):
            r = g * 16 + m
            rowsv[r, :] = rowsv[r, :] * _lane_bcast(ew16, m)

        for j in range(SUB):
          pltpu.async_copy(rowsv.at[pl.ds(j * SLEN, SLEN), :],
                           acc_sh.at[dstv.at[j]], ssem, add=True)

    def _drain_scatter(q, dstv, rowsv, ssem):
      @pl.when(_valid(q))
      def _():
        for j in range(SUB):
          pltpu.make_async_copy(rowsv.at[pl.ds(j * SLEN, SLEN), :],
                                acc_sh.at[dstv.at[j]], ssem).wait()

    _load_fire(0, srcA, dstA, ewA, rowsA, isA, gsA)

    @pl.loop(0, QMAX // 3)
    def _chunk(t):
      q = 3 * t
      _drain_scatter(q - 2, dstB, rowsB, ssB)
      _load_fire(q + 1, srcB, dstB, ewB, rowsB, isB, gsB)
      _mul_scatter(q, srcA, dstA, ewA, rowsA, gsA, ssA)
      _drain_scatter(q - 1, dstC, rowsC, ssC)
      _load_fire(q + 2, srcC, dstC, ewC, rowsC, isC, gsC)
      _mul_scatter(q + 1, srcB, dstB, ewB, rowsB, gsB, ssB)
      _drain_scatter(q, dstA, rowsA, ssA)
      _load_fire(q + 3, srcA, dstA, ewA, rowsA, isA, gsA)
      _mul_scatter(q + 2, srcC, dstC, ewC, rowsC, gsC, ssC)

    _drain_scatter(QMAX - 2, dstB, rowsB, ssB)
    _drain_scatter(QMAX - 1, dstC, rowsC, ssC)

  pl.run_scoped(
      _run,
      pltpu.VMEM((SUB, SLEN), jnp.int32),
      pltpu.VMEM((SUB, SLEN), jnp.int32),
      pltpu.VMEM((SUB, SLEN), jnp.float32),
      pltpu.VMEM((CHUNK, F), jnp.float32),
      pltpu.SemaphoreType.DMA,
      pltpu.SemaphoreType.DMA,
      pltpu.SemaphoreType.DMA,
      pltpu.VMEM((SUB, SLEN), jnp.int32),
      pltpu.VMEM((SUB, SLEN), jnp.int32),
      pltpu.VMEM((SUB, SLEN), jnp.float32),
      pltpu.VMEM((CHUNK, F), jnp.float32),
      pltpu.SemaphoreType.DMA,
      pltpu.SemaphoreType.DMA,
      pltpu.SemaphoreType.DMA,
      pltpu.VMEM((SUB, SLEN), jnp.int32),
      pltpu.VMEM((SUB, SLEN), jnp.int32),
      pltpu.VMEM((SUB, SLEN), jnp.float32),
      pltpu.VMEM((CHUNK, F), jnp.float32),
      pltpu.SemaphoreType.DMA,
      pltpu.SemaphoreType.DMA,
      pltpu.SemaphoreType.DMA,
  )

  plsc.subcore_barrier()

  @pl.loop(0, 24)
  def _co(b):
    pltpu.sync_copy(acc_sh.at[pl.ds(s * SA + b * CB, CB), :],
                    out_hbm.at[c, pl.ds(s * SA + b * CB, CB), :])

  @pl.when(s < 15)
  def _():
    pltpu.sync_copy(acc_sh.at[pl.ds(s * SA + 24 * CB, SA - 24 * CB), :],
                    out_hbm.at[c, pl.ds(s * SA + 24 * CB, SA - 24 * CB), :])
  @pl.when(s == 15)
  def _():
    pltpu.sync_copy(acc_sh.at[pl.ds(15 * SA + 24 * CB, SLAST - 24 * CB), :],
                    out_hbm.at[c, pl.ds(15 * SA + 24 * CB, SLAST - 24 * CB), :])


PB = 128
PBLK = 25
NP = NW * PBLK * PB
TAIL_BASE = 31 * PBLK * PB + 6 * PB
TAIL = N - TAIL_BASE


@functools.partial(
    pl.kernel,
    out_type=jax.ShapeDtypeStruct((NC, G, F), jnp.float32),
    mesh=_mesh,
    compiler_params=pltpu.CompilerParams(use_tc_tiling_on_sc=False),
    scratch_types=[
        pltpu.VMEM((PBLK, PB), jnp.int32),
        pltpu.VMEM((1, TAIL), jnp.int32),
        pltpu.VMEM((PB, F), jnp.float32),
        pltpu.VMEM((PB, F), jnp.float32),
        pltpu.VMEM((PB, F), jnp.float32),
        pltpu.VMEM((1, F), jnp.float32),
        pltpu.VMEM_SHARED((G, F), jnp.float32),
    ],
)
def _pool(p3_hbm, t3_hbm, b3_hbm, batch_hbm, out_hbm,
          bidx_v, tidx_v, prow_v, paux_v, taux_v, bv, pool_sh):
  c = lax.axis_index("c")
  s = lax.axis_index("s")
  wid = c * NS + s

  @pl.when(s == 0)
  def _():
    def _zero(zbuf):
      @plsc.parallel_loop(0, G, unroll=4)
      def _(r):
        zbuf[r, :] = jnp.zeros((F,), jnp.float32)
      pltpu.sync_copy(zbuf, pool_sh)
    pl.run_scoped(_zero, pltpu.VMEM((G, F), jnp.float32))
  plsc.subcore_barrier()

  pltpu.sync_copy(batch_hbm.at[wid], bidx_v)
  pltpu.sync_copy(b3_hbm, bv)
  nblk = jnp.where(wid < 31, PBLK, 6)

  def _epilogue(nrows, nb, idx):
    pltpu.sync_copy(p3_hbm.at[0, pl.ds(nb, nrows), :],
                    prow_v.at[pl.ds(0, nrows), :])
    pltpu.sync_copy(p3_hbm.at[1, pl.ds(nb, nrows), :],
                    paux_v.at[pl.ds(0, nrows), :])
    pltpu.sync_copy(t3_hbm.at[pl.ds(nb, nrows), :],
                    taux_v.at[pl.ds(0, nrows), :])

    @plsc.parallel_loop(0, nrows, unroll=4)
    def _(r):
      prow_v[r, :] = jnp.maximum(
          prow_v[r, :] + paux_v[r, :] + taux_v[r, :] + bv[0, :], 0.0)

    pltpu.sync_copy(prow_v.at[pl.ds(0, nrows), :], pool_sh.at[idx], add=True)

  @pl.loop(0, PBLK)
  def _blk(k):
    @pl.when(k < nblk)
    def _():
      _epilogue(PB, wid * PBLK * PB + k * PB, bidx_v.at[k])

  @pl.when(wid == 31)
  def _():
    pltpu.sync_copy(batch_hbm.at[31, pl.ds(6, 1), pl.ds(0, TAIL)], tidx_v)
    _epilogue(TAIL, TAIL_BASE, tidx_v.at[0])

  plsc.subcore_barrier()

  @pl.when(s == 0)
  def _():
    pltpu.sync_copy(pool_sh, out_hbm.at[c])


BR = 10000
GRID = N // BR


def _dense1_body(x_ref, w_ref, a_ref, o_ref):
  o_ref[...] = (jnp.dot(x_ref[...], w_ref[...],
                        preferred_element_type=jnp.float32) + a_ref[...])


def _dense1(x, w1p, e7):
  return pl.pallas_call(
      _dense1_body,
      grid=(GRID,),
      in_specs=[
          pl.BlockSpec((BR, 3), lambda i: (i, 0)),
          pl.BlockSpec((3, F), lambda i: (0, 0)),
          pl.BlockSpec((1, F), lambda i: (0, 0)),
      ],
      out_specs=pl.BlockSpec((BR, F), lambda i: (i, 0)),
      out_shape=jax.ShapeDtypeStruct((N, F), jnp.float32),
  )(x, w1p, e7)


def _dense2_body(p_ref, t1_ref, w2_ref, b1_ref, t2_ref, dis_ref):
  acc = p_ref[0] + p_ref[1]
  deg = acc[:, 7:8]
  dis = jnp.where(deg > 0, lax.rsqrt(jnp.where(deg > 0, deg, 1.0)), 0.0)
  out1 = jnp.maximum(acc + t1_ref[...] + b1_ref[...], 0.0)
  t2_ref[...] = dis * jnp.dot(out1, w2_ref[...],
                              preferred_element_type=jnp.float32)
  dis_ref[...] = dis


def _dense2(p1, t1, w2p, b1p):
  return pl.pallas_call(
      _dense2_body,
      grid=(GRID,),
      in_specs=[
          pl.BlockSpec((2, BR, F), lambda i: (0, i, 0)),
          pl.BlockSpec((BR, F), lambda i: (i, 0)),
          pl.BlockSpec((F, F), lambda i: (0, 0)),
          pl.BlockSpec((1, F), lambda i: (0, 0)),
      ],
      out_specs=[
          pl.BlockSpec((BR, F), lambda i: (i, 0)),
          pl.BlockSpec((BR, 1), lambda i: (i, 0)),
      ],
      out_shape=[
          jax.ShapeDtypeStruct((N, F), jnp.float32),
          jax.ShapeDtypeStruct((N, 1), jnp.float32),
      ],
  )(p1, t1, w2p, b1p)


def _dense3_body(p_ref, dis_ref, w3_ref, b2_ref, t3_ref):
  acc = p_ref[0] + p_ref[1]
  out2 = jnp.maximum(dis_ref[...] * acc + b2_ref[...], 0.0)
  t3_ref[...] = jnp.dot(out2, w3_ref[...], preferred_element_type=jnp.float32)


def _dense3(p2, dis, w3p, b2p):
  return pl.pallas_call(
      _dense3_body,
      grid=(GRID,),
      in_specs=[
          pl.BlockSpec((2, BR, F), lambda i: (0, i, 0)),
          pl.BlockSpec((BR, 1), lambda i: (i, 0)),
          pl.BlockSpec((F, F), lambda i: (0, 0)),
          pl.BlockSpec((1, F), lambda i: (0, 0)),
      ],
      out_specs=pl.BlockSpec((BR, F), lambda i: (i, 0)),
      out_shape=jax.ShapeDtypeStruct((N, F), jnp.float32),
  )(p2, dis, w3p, b2p)


def _dense4_body(p_ref, t3_ref, b3_ref, o_ref):
  o_ref[...] = jnp.maximum(p_ref[0] + p_ref[1] + t3_ref[...] + b3_ref[...],
                           0.0)


def _dense4(p3, t3, b3p):
  return pl.pallas_call(
      _dense4_body,
      grid=(GRID,),
      in_specs=[
          pl.BlockSpec((2, BR, F), lambda i: (0, i, 0)),
          pl.BlockSpec((BR, F), lambda i: (i, 0)),
          pl.BlockSpec((1, F), lambda i: (0, 0)),
      ],
      out_specs=pl.BlockSpec((BR, F), lambda i: (i, 0)),
      out_shape=jax.ShapeDtypeStruct((N, F), jnp.float32),
  )(p3, t3, b3p)


def _final_body(pp_ref, o_ref):
  pooled = pp_ref[0] + pp_ref[1]
  col = lax.broadcasted_iota(jnp.int32, (G, F), 1)
  neg = jnp.where(col < 6, pooled, -jnp.inf)
  m = jnp.max(neg, axis=1, keepdims=True)
  e = jnp.where(col < 6, jnp.exp(neg - m), 0.0)
  lse = jnp.log(jnp.sum(e, axis=1, keepdims=True))
  o_ref[...] = (pooled - m - lse)[:, :6]


def _final(pp):
  return pl.pallas_call(
      _final_body,
      out_shape=jax.ShapeDtypeStruct((G, 6), jnp.float32),
  )(pp)


def kernel(x, edge_index, batch, edge_weight, W1, b1, W2, b2, W3, b3):
  src = edge_index[0].astype(jnp.int32).reshape(E // SLEN, SLEN)
  dst = edge_index[1].astype(jnp.int32).reshape(E // SLEN, SLEN)
  ew2 = edge_weight.reshape(E // SLEN, SLEN)
  batchi = jnp.pad(batch.astype(jnp.int32), (0, NP - N),
                   constant_values=G - 1).reshape(NW, PBLK, PB)

  w1p = jnp.zeros((3, F), jnp.float32).at[:, :7].set(W1)
  e7 = jnp.zeros((1, F), jnp.float32).at[0, 7].set(1.0)
  b1p = jnp.zeros((1, F), jnp.float32).at[0, :7].set(b1)
  w2p = jnp.zeros((F, F), jnp.float32).at[:7, :9].set(W2)
  b2p = jnp.zeros((1, F), jnp.float32).at[0, :9].set(b2)
  w3p = jnp.zeros((F, F), jnp.float32).at[:9, :6].set(W3)
  b3p = jnp.zeros((1, F), jnp.float32).at[0, :6].set(b3)

  t1 = _dense1(x, w1p, e7)
  p1 = _edge_pass(src, dst, ew2, t1)
  t2, dis = _dense2(p1, t1, w2p, b1p)
  p2 = _edge_pass(src, dst, ew2, t2)
  t3 = _dense3(p2, dis, w3p, b2p)
  p3 = _edge_pass(src, dst, ew2, t3)
  pp = _pool(p3, t3, b3p, batchi)
  return _final(pp)


# speedup vs baseline: 125.0948x; 1.0282x over previous
import functools

import jax
import jax.numpy as jnp
from jax import lax
from jax.experimental import pallas as pl
from jax.experimental.pallas import tpu as pltpu
from jax.experimental.pallas import tpu_sc as plsc

N = 100000
E = 6400000
G = 512
F = 16
NC = 2
NS = 16
NW = NC * NS
SLEN = 64
SUB = 8
CHUNK = SLEN * SUB
NCH = E // CHUNK
QMAX = (-(-NCH // NW) + 2) // 3 * 3
GSH = 2
SA = 6256
SLAST = N - 15 * SA
ZB = 128
CB = 256

_mesh = plsc.VectorSubcoreMesh(core_axis_name="c", subcore_axis_name="s")


def _lane_bcast(v16, m):
  return jnp.take_along_axis(v16, jnp.full((16,), m, jnp.int32), axis=0)


@functools.partial(
    pl.kernel,
    out_type=jax.ShapeDtypeStruct((NC, N, F), jnp.float32),
    mesh=_mesh,
    compiler_params=pltpu.CompilerParams(use_tc_tiling_on_sc=False),
    scratch_types=[
        pltpu.VMEM_SHARED((N, F), jnp.float32),
    ],
)
def _edge_pass(src_hbm, dst_hbm, ew_hbm, table_hbm, out_hbm, acc_sh):
  c = lax.axis_index("c")
  s = lax.axis_index("s")
  wid = c * NS + s

  def _zero(zbuf):
    @plsc.parallel_loop(0, ZB, unroll=4)
    def _(r):
      zbuf[r, :] = jnp.zeros((F,), jnp.float32)
    base = s * SA

    @pl.loop(0, 48)
    def _zc(b):
      pltpu.sync_copy(zbuf, acc_sh.at[pl.ds(base + b * ZB, ZB), :])

    @pl.when(s < 15)
    def _():
      pltpu.sync_copy(zbuf.at[pl.ds(0, SA - 48 * ZB), :],
                      acc_sh.at[pl.ds(base + 48 * ZB, SA - 48 * ZB), :])
    @pl.when(s == 15)
    def _():
      pltpu.sync_copy(zbuf.at[pl.ds(0, SLAST - 48 * ZB), :],
                      acc_sh.at[pl.ds(base + 48 * ZB, SLAST - 48 * ZB), :])

  pl.run_scoped(_zero, pltpu.VMEM((ZB, F), jnp.float32))
  plsc.subcore_barrier()

  def _run(srcA, dstA, ewA, rowsA, isA, gsA, ssA,
           srcB, dstB, ewB, rowsB, isB, gsB, ssB,
           srcC, dstC, ewC, rowsC, isC, gsC, ssC):
    def _valid(q):
      cid = q * NW + wid
      return (cid >= 0) & (cid < NCH)

    def _load_fire(q, srcv, dstv, ewv, rowsv, isem, gsem):
      @pl.when(_valid(q))
      def _():
        rbase = (q * NW + wid) * SUB
        pltpu.async_copy(src_hbm.at[pl.ds(rbase, SUB), :], srcv, isem)
        pltpu.async_copy(dst_hbm.at[pl.ds(rbase, SUB), :], dstv, isem)
        pltpu.async_copy(ew_hbm.at[pl.ds(rbase, SUB), :], ewv, isem)
        pltpu.make_async_copy(src_hbm.at[pl.ds(rbase, SUB), :], srcv,
                              isem).wait()
        for j in range(SUB):
          pltpu.async_copy(table_hbm.at[srcv.at[j]],
                           rowsv.at[pl.ds(j * SLEN, SLEN), :], gsem)

    def _mul_scatter(q, srcv, dstv, ewv, rowsv, isem, gsem, ssem):
      @pl.when(_valid(q))
      def _():
        rbase = (q * NW + wid) * SUB
        pltpu.make_async_copy(dst_hbm.at[pl.ds(rbase, SUB), :], dstv,
                              isem).wait()
        pltpu.make_async_copy(ew_hbm.at[pl.ds(rbase, SUB), :], ewv,
                              isem).wait()
        for j in range(SUB):
          pltpu.make_async_copy(table_hbm.at[srcv.at[j]],
                                rowsv.at[pl.ds(j * SLEN, SLEN), :],
                                gsem).wait()

        @plsc.parallel_loop(0, CHUNK // 16, unroll=2)
        def _mul(g):
          j = lax.shift_right_logical(g, GSH)
          k = lax.bitwise_and(g, (1 << GSH) - 1)
          ew16 = ewv[j, pl.ds(k * 16, 16)]
          for m in range(16):
            r = g * 16 + m
            rowsv[r, :] = rowsv[r, :] * _lane_bcast(ew16, m)

        for j in range(SUB):
          pltpu.async_copy(rowsv.at[pl.ds(j * SLEN, SLEN), :],
                           acc_sh.at[dstv.at[j]], ssem, add=True)

    def _drain_scatter(q, dstv, rowsv, ssem):
      @pl.when(_valid(q))
      def _():
        for j in range(SUB):
          pltpu.make_async_copy(rowsv.at[pl.ds(j * SLEN, SLEN), :],
                                acc_sh.at[dstv.at[j]], ssem).wait()

    _load_fire(0, srcA, dstA, ewA, rowsA, isA, gsA)

    @pl.loop(0, QMAX // 3)
    def _chunk(t):
      q = 3 * t
      _drain_scatter(q - 2, dstB, rowsB, ssB)
      _load_fire(q + 1, srcB, dstB, ewB, rowsB, isB, gsB)
      _mul_scatter(q, srcA, dstA, ewA, rowsA, isA, gsA, ssA)
      _drain_scatter(q - 1, dstC, rowsC, ssC)
      _load_fire(q + 2, srcC, dstC, ewC, rowsC, isC, gsC)
      _mul_scatter(q + 1, srcB, dstB, ewB, rowsB, isB, gsB, ssB)
      _drain_scatter(q, dstA, rowsA, ssA)
      _load_fire(q + 3, srcA, dstA, ewA, rowsA, isA, gsA)
      _mul_scatter(q + 2, srcC, dstC, ewC, rowsC, isC, gsC, ssC)

    _drain_scatter(QMAX - 2, dstB, rowsB, ssB)
    _drain_scatter(QMAX - 1, dstC, rowsC, ssC)

  pl.run_scoped(
      _run,
      pltpu.VMEM((SUB, SLEN), jnp.int32),
      pltpu.VMEM((SUB, SLEN), jnp.int32),
      pltpu.VMEM((SUB, SLEN), jnp.float32),
      pltpu.VMEM((CHUNK, F), jnp.float32),
      pltpu.SemaphoreType.DMA,
      pltpu.SemaphoreType.DMA,
      pltpu.SemaphoreType.DMA,
      pltpu.VMEM((SUB, SLEN), jnp.int32),
      pltpu.VMEM((SUB, SLEN), jnp.int32),
      pltpu.VMEM((SUB, SLEN), jnp.float32),
      pltpu.VMEM((CHUNK, F), jnp.float32),
      pltpu.SemaphoreType.DMA,
      pltpu.SemaphoreType.DMA,
      pltpu.SemaphoreType.DMA,
      pltpu.VMEM((SUB, SLEN), jnp.int32),
      pltpu.VMEM((SUB, SLEN), jnp.int32),
      pltpu.VMEM((SUB, SLEN), jnp.float32),
      pltpu.VMEM((CHUNK, F), jnp.float32),
      pltpu.SemaphoreType.DMA,
      pltpu.SemaphoreType.DMA,
      pltpu.SemaphoreType.DMA,
  )

  plsc.subcore_barrier()

  @pl.loop(0, 24)
  def _co(b):
    pltpu.sync_copy(acc_sh.at[pl.ds(s * SA + b * CB, CB), :],
                    out_hbm.at[c, pl.ds(s * SA + b * CB, CB), :])

  @pl.when(s < 15)
  def _():
    pltpu.sync_copy(acc_sh.at[pl.ds(s * SA + 24 * CB, SA - 24 * CB), :],
                    out_hbm.at[c, pl.ds(s * SA + 24 * CB, SA - 24 * CB), :])
  @pl.when(s == 15)
  def _():
    pltpu.sync_copy(acc_sh.at[pl.ds(15 * SA + 24 * CB, SLAST - 24 * CB), :],
                    out_hbm.at[c, pl.ds(15 * SA + 24 * CB, SLAST - 24 * CB), :])


PB = 128
PBLK = 25
NP = NW * PBLK * PB
TAIL_BASE = 31 * PBLK * PB + 6 * PB
TAIL = N - TAIL_BASE


@functools.partial(
    pl.kernel,
    out_type=jax.ShapeDtypeStruct((NC, G, F), jnp.float32),
    mesh=_mesh,
    compiler_params=pltpu.CompilerParams(use_tc_tiling_on_sc=False),
    scratch_types=[
        pltpu.VMEM((PBLK, PB), jnp.int32),
        pltpu.VMEM((1, TAIL), jnp.int32),
        pltpu.VMEM((PB, F), jnp.float32),
        pltpu.VMEM((PB, F), jnp.float32),
        pltpu.VMEM((PB, F), jnp.float32),
        pltpu.VMEM((1, F), jnp.float32),
        pltpu.VMEM_SHARED((G, F), jnp.float32),
    ],
)
def _pool(p3_hbm, t3_hbm, b3_hbm, batch_hbm, out_hbm,
          bidx_v, tidx_v, prow_v, paux_v, taux_v, bv, pool_sh):
  c = lax.axis_index("c")
  s = lax.axis_index("s")
  wid = c * NS + s

  @pl.when(s == 0)
  def _():
    def _zero(zbuf):
      @plsc.parallel_loop(0, G, unroll=4)
      def _(r):
        zbuf[r, :] = jnp.zeros((F,), jnp.float32)
      pltpu.sync_copy(zbuf, pool_sh)
    pl.run_scoped(_zero, pltpu.VMEM((G, F), jnp.float32))
  plsc.subcore_barrier()

  pltpu.sync_copy(batch_hbm.at[wid], bidx_v)
  pltpu.sync_copy(b3_hbm, bv)
  nblk = jnp.where(wid < 31, PBLK, 6)

  def _epilogue(nrows, nb, idx):
    pltpu.sync_copy(p3_hbm.at[0, pl.ds(nb, nrows), :],
                    prow_v.at[pl.ds(0, nrows), :])
    pltpu.sync_copy(p3_hbm.at[1, pl.ds(nb, nrows), :],
                    paux_v.at[pl.ds(0, nrows), :])
    pltpu.sync_copy(t3_hbm.at[pl.ds(nb, nrows), :],
                    taux_v.at[pl.ds(0, nrows), :])

    @plsc.parallel_loop(0, nrows, unroll=4)
    def _(r):
      prow_v[r, :] = jnp.maximum(
          prow_v[r, :] + paux_v[r, :] + taux_v[r, :] + bv[0, :], 0.0)

    pltpu.sync_copy(prow_v.at[pl.ds(0, nrows), :], pool_sh.at[idx], add=True)

  @pl.loop(0, PBLK)
  def _blk(k):
    @pl.when(k < nblk)
    def _():
      _epilogue(PB, wid * PBLK * PB + k * PB, bidx_v.at[k])

  @pl.when(wid == 31)
  def _():
    pltpu.sync_copy(batch_hbm.at[31, pl.ds(6, 1), pl.ds(0, TAIL)], tidx_v)
    _epilogue(TAIL, TAIL_BASE, tidx_v.at[0])

  plsc.subcore_barrier()

  @pl.when(s == 0)
  def _():
    pltpu.sync_copy(pool_sh, out_hbm.at[c])


BR = 10000
GRID = N // BR


def _dense1_body(x_ref, w_ref, a_ref, o_ref):
  o_ref[...] = (jnp.dot(x_ref[...], w_ref[...],
                        preferred_element_type=jnp.float32) + a_ref[...])


def _dense1(x, w1p, e7):
  return pl.pallas_call(
      _dense1_body,
      grid=(GRID,),
      in_specs=[
          pl.BlockSpec((BR, 3), lambda i: (i, 0)),
          pl.BlockSpec((3, F), lambda i: (0, 0)),
          pl.BlockSpec((1, F), lambda i: (0, 0)),
      ],
      out_specs=pl.BlockSpec((BR, F), lambda i: (i, 0)),
      out_shape=jax.ShapeDtypeStruct((N, F), jnp.float32),
  )(x, w1p, e7)


def _dense2_body(p_ref, t1_ref, w2_ref, b1_ref, t2_ref, dis_ref):
  acc = p_ref[0] + p_ref[1]
  deg = acc[:, 7:8]
  dis = jnp.where(deg > 0, lax.rsqrt(jnp.where(deg > 0, deg, 1.0)), 0.0)
  out1 = jnp.maximum(acc + t1_ref[...] + b1_ref[...], 0.0)
  t2_ref[...] = dis * jnp.dot(out1, w2_ref[...],
                              preferred_element_type=jnp.float32)
  dis_ref[...] = dis


def _dense2(p1, t1, w2p, b1p):
  return pl.pallas_call(
      _dense2_body,
      grid=(GRID,),
      in_specs=[
          pl.BlockSpec((2, BR, F), lambda i: (0, i, 0)),
          pl.BlockSpec((BR, F), lambda i: (i, 0)),
          pl.BlockSpec((F, F), lambda i: (0, 0)),
          pl.BlockSpec((1, F), lambda i: (0, 0)),
      ],
      out_specs=[
          pl.BlockSpec((BR, F), lambda i: (i, 0)),
          pl.BlockSpec((BR, 1), lambda i: (i, 0)),
      ],
      out_shape=[
          jax.ShapeDtypeStruct((N, F), jnp.float32),
          jax.ShapeDtypeStruct((N, 1), jnp.float32),
      ],
  )(p1, t1, w2p, b1p)


def _dense3_body(p_ref, dis_ref, w3_ref, b2_ref, t3_ref):
  acc = p_ref[0] + p_ref[1]
  out2 = jnp.maximum(dis_ref[...] * acc + b2_ref[...], 0.0)
  t3_ref[...] = jnp.dot(out2, w3_ref[...], preferred_element_type=jnp.float32)


def _dense3(p2, dis, w3p, b2p):
  return pl.pallas_call(
      _dense3_body,
      grid=(GRID,),
      in_specs=[
          pl.BlockSpec((2, BR, F), lambda i: (0, i, 0)),
          pl.BlockSpec((BR, 1), lambda i: (i, 0)),
          pl.BlockSpec((F, F), lambda i: (0, 0)),
          pl.BlockSpec((1, F), lambda i: (0, 0)),
      ],
      out_specs=pl.BlockSpec((BR, F), lambda i: (i, 0)),
      out_shape=jax.ShapeDtypeStruct((N, F), jnp.float32),
  )(p2, dis, w3p, b2p)


def _dense4_body(p_ref, t3_ref, b3_ref, o_ref):
  o_ref[...] = jnp.maximum(p_ref[0] + p_ref[1] + t3_ref[...] + b3_ref[...],
                           0.0)


def _dense4(p3, t3, b3p):
  return pl.pallas_call(
      _dense4_body,
      grid=(GRID,),
      in_specs=[
          pl.BlockSpec((2, BR, F), lambda i: (0, i, 0)),
          pl.BlockSpec((BR, F), lambda i: (i, 0)),
          pl.BlockSpec((1, F), lambda i: (0, 0)),
      ],
      out_specs=pl.BlockSpec((BR, F), lambda i: (i, 0)),
      out_shape=jax.ShapeDtypeStruct((N, F), jnp.float32),
  )(p3, t3, b3p)


def _final_body(pp_ref, o_ref):
  pooled = pp_ref[0] + pp_ref[1]
  col = lax.broadcasted_iota(jnp.int32, (G, F), 1)
  neg = jnp.where(col < 6, pooled, -jnp.inf)
  m = jnp.max(neg, axis=1, keepdims=True)
  e = jnp.where(col < 6, jnp.exp(neg - m), 0.0)
  lse = jnp.log(jnp.sum(e, axis=1, keepdims=True))
  o_ref[...] = (pooled - m - lse)[:, :6]


def _final(pp):
  return pl.pallas_call(
      _final_body,
      out_shape=jax.ShapeDtypeStruct((G, 6), jnp.float32),
  )(pp)


def kernel(x, edge_index, batch, edge_weight, W1, b1, W2, b2, W3, b3):
  src = edge_index[0].astype(jnp.int32).reshape(E // SLEN, SLEN)
  dst = edge_index[1].astype(jnp.int32).reshape(E // SLEN, SLEN)
  ew2 = edge_weight.reshape(E // SLEN, SLEN)
  batchi = jnp.pad(batch.astype(jnp.int32), (0, NP - N),
                   constant_values=G - 1).reshape(NW, PBLK, PB)

  w1p = jnp.zeros((3, F), jnp.float32).at[:, :7].set(W1)
  e7 = jnp.zeros((1, F), jnp.float32).at[0, 7].set(1.0)
  b1p = jnp.zeros((1, F), jnp.float32).at[0, :7].set(b1)
  w2p = jnp.zeros((F, F), jnp.float32).at[:7, :9].set(W2)
  b2p = jnp.zeros((1, F), jnp.float32).at[0, :9].set(b2)
  w3p = jnp.zeros((F, F), jnp.float32).at[:9, :6].set(W3)
  b3p = jnp.zeros((1, F), jnp.float32).at[0, :6].set(b3)

  t1 = _dense1(x, w1p, e7)
  p1 = _edge_pass(src, dst, ew2, t1)
  t2, dis = _dense2(p1, t1, w2p, b1p)
  p2 = _edge_pass(src, dst, ew2, t2)
  t3 = _dense3(p2, dis, w3p, b2p)
  p3 = _edge_pass(src, dst, ew2, t3)
  pp = _pool(p3, t3, b3p, batchi)
  return _final(pp)
